# baseline (speedup 1.0000x reference)
import jax
import jax.numpy as jnp
from jax import lax
from jax.experimental import pallas as pl
from jax.experimental.pallas import tpu as pltpu
from jax.experimental.pallas import tpu_sc as plsc

N_NODES = 10000
N_EDGES = 320000
D_IN = 128
N_HEADS = 8
D_HEAD = 16
D_OUT = N_HEADS * D_HEAD

NC = 2
NS = 16
NW = NC * NS
EPW = N_EDGES // NW
CHUNK = 80
NCHUNK = EPW // CHUNK
CHUNK_B = 400
NCHUNK_B = EPW // CHUNK_B
ROWS_PER_TILE = 624
ROWS_TAIL = N_NODES - NS * ROWS_PER_TILE


def _lin_kernel(feat_ref, wt_ref, b1_ref, b2_ref, fs_ref, elr_ref, rel_ref):
    x = feat_ref[...]
    fs = jnp.dot(x, wt_ref[...], preferred_element_type=jnp.float32,
                 precision=lax.Precision.HIGHEST)
    fs_ref[...] = fs
    elr_ref[...] = jnp.dot(fs, b1_ref[...], preferred_element_type=jnp.float32,
                           precision=lax.Precision.HIGHEST)
    rel_ref[...] = jnp.dot(fs, b2_ref[...], preferred_element_type=jnp.float32,
                           precision=lax.Precision.HIGHEST)


def _linear_stage(feat, wt, b1, b2):
    blk = 1000
    grid = N_NODES // blk
    return pl.pallas_call(
        _lin_kernel,
        grid=(grid,),
        in_specs=[
            pl.BlockSpec((blk, D_IN), lambda i: (i, 0)),
            pl.BlockSpec((D_IN, D_OUT), lambda i: (0, 0)),
            pl.BlockSpec((D_OUT, 16), lambda i: (0, 0)),
            pl.BlockSpec((D_OUT, 16), lambda i: (0, 0)),
        ],
        out_specs=[
            pl.BlockSpec((blk, D_OUT), lambda i: (i, 0)),
            pl.BlockSpec((blk, 16), lambda i: (i, 0)),
            pl.BlockSpec((blk, 16), lambda i: (i, 0)),
        ],
        out_shape=[
            jax.ShapeDtypeStruct((N_NODES, D_OUT), jnp.float32),
            jax.ShapeDtypeStruct((N_NODES, 16), jnp.float32),
            jax.ShapeDtypeStruct((N_NODES, 16), jnp.float32),
        ],
    )(feat, wt, b1, b2)


def _rot8(v):
    perm = (lax.iota(jnp.int32, 16) + 8) % 16
    return lax.gather(
        v, perm[:, None],
        dimension_numbers=lax.GatherDimensionNumbers(
            offset_dims=(), collapsed_slice_dims=(0,), start_index_map=(0,)),
        slice_sizes=(1,), mode=lax.GatherScatterMode.PROMISE_IN_BOUNDS)


def _agg_body(fs_hbm, elr_hbm, rel_hbm, src_hbm, dst_hbm,
              es0_out, es1_out, rst0_out, rst1_out, ee_out, *scr):
    SRC = scr[0:3]
    DST = scr[3:6]
    RS = scr[6:8]
    RD = scr[8:10]
    EE = scr[10:12]
    EEP = scr[12:14]
    FSR = scr[14:17]
    esum_sh, rst_sh = scr[17], scr[18]
    gRS = scr[19:21]
    gRD = scr[21:23]
    gF = scr[23:26]
    sE = scr[26:28]
    sR = scr[28:31]
    sP = scr[31:33]

    cid = lax.axis_index("c")
    sid = lax.axis_index("s")
    wid = cid * NS + sid

    def _zero_fsr(i, _):
        e = i // N_HEADS
        h = i % N_HEADS
        FSR[0][e, pl.ds(h * D_HEAD, D_HEAD)] = jnp.zeros((16,), jnp.float32)
        return 0
    lax.fori_loop(0, CHUNK * N_HEADS, _zero_fsr, 0)

    def _zero_ee(i, _):
        EE[0][i] = jnp.zeros((16,), jnp.float32)
        return 0
    lax.fori_loop(0, CHUNK, _zero_ee, 0)

    r0 = sid * ROWS_PER_TILE
    for off in range(0, ROWS_PER_TILE, CHUNK):
        n = min(CHUNK, ROWS_PER_TILE - off)
        pltpu.sync_copy(FSR[0].at[pl.ds(0, n)], rst_sh.at[pl.ds(r0 + off, n)])
        pltpu.sync_copy(EE[0].at[pl.ds(0, n)], esum_sh.at[pl.ds(r0 + off, n)])

    @pl.when(sid == NS - 1)
    def _zero_tail():
        t0 = NS * ROWS_PER_TILE
        pltpu.sync_copy(FSR[0].at[pl.ds(0, ROWS_TAIL)],
                        rst_sh.at[pl.ds(t0, ROWS_TAIL)])
        pltpu.sync_copy(EE[0].at[pl.ds(0, ROWS_TAIL)],
                        esum_sh.at[pl.ds(t0, ROWS_TAIL)])

    plsc.subcore_barrier()

    mask = lax.iota(jnp.int32, 16) < N_HEADS

    def _issue(kk, p):
        si, s, f = p % 3, p % 2, p % 3
        base = wid * EPW + kk * CHUNK
        pltpu.sync_copy(src_hbm.at[pl.ds(base, CHUNK)], SRC[si])
        pltpu.sync_copy(dst_hbm.at[pl.ds(base, CHUNK)], DST[si])
        pltpu.async_copy(elr_hbm.at[SRC[si]], RS[s], gRS[s])
        pltpu.async_copy(rel_hbm.at[DST[si]], RD[s], gRD[s])
        pltpu.async_copy(fs_hbm.at[SRC[si]], FSR[f], gF[f])

    def _wait_se(p):
        si, s = p % 3, p % 2
        pltpu.make_async_copy(EE[s], esum_sh.at[DST[si]], sE[s]).wait()

    def _wait_sr(p):
        f = p % 3
        pltpu.make_async_copy(FSR[f], rst_sh.at[DST[f]], sR[f]).wait()

    def _wait_sp(kk, p):
        s = p % 2
        rbase = (wid * EPW + kk * CHUNK) // 2
        pltpu.make_async_copy(EEP[s], ee_out.at[pl.ds(rbase, CHUNK // 2)],
                              sP[s]).wait()

    def _process(kk, p):
        si, s, f = p % 3, p % 2, p % 3
        pltpu.make_async_copy(elr_hbm.at[SRC[si]], RS[s], gRS[s]).wait()
        pltpu.make_async_copy(rel_hbm.at[DST[si]], RD[s], gRD[s]).wait()
        pltpu.make_async_copy(fs_hbm.at[SRC[si]], FSR[f], gF[f]).wait()

        @plsc.parallel_loop(0, CHUNK, step=1, unroll=2)
        def _edge(e):
            v = RS[s][e] + RD[s][e]
            v = jnp.where(v >= 0, v, 0.2 * v)
            ee = jnp.where(mask, jnp.exp(v), 0.0)
            EE[s][e] = ee
            for h in range(N_HEADS):
                sc = ee[h]
                FSR[f][e, pl.ds(h * D_HEAD, D_HEAD)] = (
                    FSR[f][e, pl.ds(h * D_HEAD, D_HEAD)] * sc)

        pltpu.async_copy(EE[s], esum_sh.at[DST[si]], sE[s], add=True)

        @plsc.parallel_loop(0, CHUNK // 2, step=1, unroll=2)
        def _pair(i):
            EEP[s][i] = EE[s][2 * i] + _rot8(EE[s][2 * i + 1])

        rbase = (wid * EPW + kk * CHUNK) // 2
        pltpu.async_copy(EEP[s], ee_out.at[pl.ds(rbase, CHUNK // 2)], sP[s])
        pltpu.async_copy(FSR[f], rst_sh.at[DST[si]], sR[f], add=True)

    _issue(0, 0)

    def _six(k, _):
        for j in range(6):
            kk = 6 * k + j

            @pl.when(kk >= 2)
            def _w_rst():
                _wait_sr((j + 4) % 6)

            @pl.when(kk >= 1)
            def _w_es():
                _wait_se((j + 5) % 6)
                _wait_sp(kk - 1, (j + 5) % 6)

            _issue(kk + 1, (j + 1) % 6)
            _process(kk, j)
        return 0

    n_six = (NCHUNK - 5) // 6
    lax.fori_loop(0, n_six, _six, 0)

    for kk in range(6 * n_six, NCHUNK):
        _wait_sr((kk - 2) % 6)
        _wait_se((kk - 1) % 6)
        _wait_sp(kk - 1, (kk - 1) % 6)
        if kk < NCHUNK - 1:
            _issue(kk + 1, (kk + 1) % 6)
        _process(kk, kk % 6)

    _wait_se((NCHUNK - 1) % 6)
    _wait_sp(NCHUNK - 1, (NCHUNK - 1) % 6)
    _wait_sr((NCHUNK - 2) % 6)
    _wait_sr((NCHUNK - 1) % 6)

    plsc.subcore_barrier()

    def _dump(esum_out, rst_out):
        pltpu.sync_copy(esum_sh.at[pl.ds(r0, ROWS_PER_TILE)],
                        esum_out.at[pl.ds(r0, ROWS_PER_TILE)])
        pltpu.sync_copy(rst_sh.at[pl.ds(r0, ROWS_PER_TILE)],
                        rst_out.at[pl.ds(r0, ROWS_PER_TILE)])

        @pl.when(sid == NS - 1)
        def _dump_tail():
            t0 = NS * ROWS_PER_TILE
            pltpu.sync_copy(esum_sh.at[pl.ds(t0, ROWS_TAIL)],
                            esum_out.at[pl.ds(t0, ROWS_TAIL)])
            pltpu.sync_copy(rst_sh.at[pl.ds(t0, ROWS_TAIL)],
                            rst_out.at[pl.ds(t0, ROWS_TAIL)])

    @pl.when(cid == 0)
    def _d0():
        _dump(es0_out, rst0_out)

    @pl.when(cid == 1)
    def _d1():
        _dump(es1_out, rst1_out)


def _aggregate_stage(fs, elr, rel, src, dst):
    mesh = plsc.VectorSubcoreMesh(core_axis_name="c", subcore_axis_name="s",
                                  num_cores=NC, num_subcores=NS)
    f = pl.kernel(
        _agg_body,
        out_type=[
            jax.ShapeDtypeStruct((N_NODES, 16), jnp.float32),
            jax.ShapeDtypeStruct((N_NODES, 16), jnp.float32),
            jax.ShapeDtypeStruct((N_NODES, D_OUT), jnp.float32),
            jax.ShapeDtypeStruct((N_NODES, D_OUT), jnp.float32),
            jax.ShapeDtypeStruct((N_EDGES // 2, 16), jnp.float32),
        ],
        mesh=mesh,
        scratch_types=(
            [pltpu.VMEM((CHUNK,), jnp.int32)] * 6
            + [pltpu.VMEM((CHUNK, 16), jnp.float32)] * 6
            + [pltpu.VMEM((CHUNK // 2, 16), jnp.float32)] * 2
            + [pltpu.VMEM((CHUNK, D_OUT), jnp.float32)] * 3
            + [pltpu.VMEM_SHARED((N_NODES, 16), jnp.float32),
               pltpu.VMEM_SHARED((N_NODES, D_OUT), jnp.float32)]
            + [pltpu.SemaphoreType.DMA] * 14
        ),
        compiler_params=pltpu.CompilerParams(use_tc_tiling_on_sc=False),
    )
    return f(fs, elr, rel, src, dst)


def _attn_body(ee_hbm, esc_hbm, dst_hbm, attn_out, *scr):
    DST = scr[0:2]
    ES = scr[2:4]
    EP = scr[4:6]
    AV = scr[6:8]
    gES = scr[8:10]
    gEP = scr[10:12]
    sA = scr[12:14]

    cid = lax.axis_index("c")
    sid = lax.axis_index("s")
    wid = cid * NS + sid
    HCH = CHUNK_B // 2

    def _issue(kk, b):
        base = wid * EPW + kk * CHUNK_B
        pltpu.sync_copy(dst_hbm.at[pl.ds(base, CHUNK_B)], DST[b])
        pltpu.async_copy(esc_hbm.at[DST[b]], ES[b], gES[b])
        pltpu.async_copy(ee_hbm.at[pl.ds(base // 2, HCH)], EP[b], gEP[b])

    def _wait_sa(kk, b):
        base = wid * EPW + kk * CHUNK_B
        pltpu.make_async_copy(AV[b], attn_out.at[pl.ds(base // 2, HCH)],
                              sA[b]).wait()

    def _process(kk, b):
        pltpu.make_async_copy(esc_hbm.at[DST[b]], ES[b], gES[b]).wait()
        base = wid * EPW + kk * CHUNK_B
        pltpu.make_async_copy(ee_hbm.at[pl.ds(base // 2, HCH)], EP[b],
                              gEP[b]).wait()

        @plsc.parallel_loop(0, HCH, step=1, unroll=2)
        def _pair(i):
            d = ES[b][2 * i] + _rot8(ES[b][2 * i + 1]) + 1e-9
            AV[b][i] = EP[b][i] / d

        pltpu.async_copy(AV[b], attn_out.at[pl.ds(base // 2, HCH)], sA[b])

    _issue(0, 0)

    def _pair(k, _):
        for j in range(2):
            kk = 2 * k + j
            b, b1 = j, 1 - j

            @pl.when(kk >= 1)
            def _w():
                _wait_sa(kk - 1, b1)

            _issue(kk + 1, b1)
            _process(kk, b)
        return 0

    n_pair = (NCHUNK_B - 1) // 2
    lax.fori_loop(0, n_pair, _pair, 0)

    for kk in range(2 * n_pair, NCHUNK_B):
        b = kk % 2
        if kk < NCHUNK_B - 1:
            _wait_sa(kk - 1, 1 - b)
            _issue(kk + 1, 1 - b)
        else:
            _wait_sa(kk - 1, 1 - b)
        _process(kk, b)
    _wait_sa(NCHUNK_B - 1, (NCHUNK_B - 1) % 2)


def _attn_stage(eep, esc, dst):
    mesh = plsc.VectorSubcoreMesh(core_axis_name="c", subcore_axis_name="s",
                                  num_cores=NC, num_subcores=NS)
    f = pl.kernel(
        _attn_body,
        out_type=jax.ShapeDtypeStruct((N_EDGES // 2, 16), jnp.float32),
        mesh=mesh,
        scratch_types=(
            [pltpu.VMEM((CHUNK_B,), jnp.int32)] * 2
            + [pltpu.VMEM((CHUNK_B, 16), jnp.float32)] * 2
            + [pltpu.VMEM((CHUNK_B // 2, 16), jnp.float32)] * 4
            + [pltpu.SemaphoreType.DMA] * 6
        ),
        compiler_params=pltpu.CompilerParams(use_tc_tiling_on_sc=False),
    )
    return f(eep, esc, dst)


def _norm_kernel(p0_ref, p1_ref, e0_ref, e1_ref, exp_ref, bias_ref,
                 out_ref, esc_ref):
    es = e0_ref[...] + e1_ref[...]
    esc_ref[...] = es
    denom = jnp.dot(es, exp_ref[...], preferred_element_type=jnp.float32,
                    precision=lax.Precision.HIGHEST) + 1e-9
    out_ref[...] = (p0_ref[...] + p1_ref[...]) / denom + bias_ref[...]


def _norm_stage(p0, p1, e0, e1, expand, bias_vec):
    blk = 1000
    grid = N_NODES // blk
    return pl.pallas_call(
        _norm_kernel,
        grid=(grid,),
        in_specs=[
            pl.BlockSpec((blk, D_OUT), lambda i: (i, 0)),
            pl.BlockSpec((blk, D_OUT), lambda i: (i, 0)),
            pl.BlockSpec((blk, 16), lambda i: (i, 0)),
            pl.BlockSpec((blk, 16), lambda i: (i, 0)),
            pl.BlockSpec((16, D_OUT), lambda i: (0, 0)),
            pl.BlockSpec((1, D_OUT), lambda i: (0, 0)),
        ],
        out_specs=[
            pl.BlockSpec((blk, D_OUT), lambda i: (i, 0)),
            pl.BlockSpec((blk, 16), lambda i: (i, 0)),
        ],
        out_shape=[
            jax.ShapeDtypeStruct((N_NODES, D_OUT), jnp.float32),
            jax.ShapeDtypeStruct((N_NODES, 16), jnp.float32),
        ],
    )(p0, p1, e0, e1, expand, bias_vec)


@jax.jit
def _run(feat, edge_index, fc_weight, attn_l, attn_r, bias_param):
    src = edge_index[0].astype(jnp.int32)
    dst = edge_index[1].astype(jnp.int32)

    al = attn_l.reshape(N_HEADS, D_HEAD)
    ar = attn_r.reshape(N_HEADS, D_HEAD)
    hf = jnp.arange(D_OUT)
    head_of = hf // D_HEAD
    lane_of = hf % D_HEAD
    onehot = (head_of[:, None] == jnp.arange(N_HEADS)[None, :]).astype(jnp.float32)
    b_l = onehot * al[head_of, lane_of][:, None]
    b_r = onehot * ar[head_of, lane_of][:, None]
    b1 = jnp.concatenate([b_l, b_r], axis=1)
    b2 = jnp.concatenate([b_r, b_l], axis=1)

    fs, elr, rel = _linear_stage(feat, fc_weight.T, b1, b2)

    es0, es1, p0, p1, eep = _aggregate_stage(fs, elr, rel, src, dst)

    expand = (jnp.arange(16)[:, None] == head_of[None, :]).astype(jnp.float32)
    bias_vec = bias_param.reshape(1, D_OUT)
    rst, esc = _norm_stage(p0, p1, es0, es1, expand, bias_vec)

    attn_pairs = _attn_stage(eep, esc, dst)

    return (rst.reshape(N_NODES, N_HEADS, D_HEAD),
            attn_pairs.reshape(N_EDGES, N_HEADS))


def kernel(feat, edge_index, e_feat, layer_idx, fc_weight, attn_l, attn_r, bias_param):
    del e_feat, layer_idx
    return _run(feat, edge_index, fc_weight, attn_l, attn_r, bias_param)


# speedup vs baseline: 1.0326x; 1.0326x over previous
import jax
import jax.numpy as jnp
from jax import lax
from jax.experimental import pallas as pl
from jax.experimental.pallas import tpu as pltpu
from jax.experimental.pallas import tpu_sc as plsc

N_NODES = 10000
N_EDGES = 320000
D_IN = 128
N_HEADS = 8
D_HEAD = 16
D_OUT = N_HEADS * D_HEAD

NC = 2
NS = 16
NW = NC * NS
EPW = N_EDGES // NW
CHUNK = 80
NCHUNK = EPW // CHUNK
CHUNK_B = 400
NCHUNK_B = EPW // CHUNK_B
ROWS_PER_TILE = 624
ROWS_TAIL = N_NODES - NS * ROWS_PER_TILE


def _lin_kernel(feat_ref, wt_ref, b1_ref, b2_ref, fs_ref, elr_ref, rel_ref):
    x = feat_ref[...]
    fs = jnp.dot(x, wt_ref[...], preferred_element_type=jnp.float32,
                 precision=lax.Precision.HIGHEST)
    fs_ref[...] = fs
    elr_ref[...] = jnp.dot(fs, b1_ref[...], preferred_element_type=jnp.float32,
                           precision=lax.Precision.HIGHEST)
    rel_ref[...] = jnp.dot(fs, b2_ref[...], preferred_element_type=jnp.float32,
                           precision=lax.Precision.HIGHEST)


def _linear_stage(feat, wt, b1, b2):
    blk = 1000
    grid = N_NODES // blk
    return pl.pallas_call(
        _lin_kernel,
        grid=(grid,),
        in_specs=[
            pl.BlockSpec((blk, D_IN), lambda i: (i, 0)),
            pl.BlockSpec((D_IN, D_OUT), lambda i: (0, 0)),
            pl.BlockSpec((D_OUT, 16), lambda i: (0, 0)),
            pl.BlockSpec((D_OUT, 16), lambda i: (0, 0)),
        ],
        out_specs=[
            pl.BlockSpec((blk, D_OUT), lambda i: (i, 0)),
            pl.BlockSpec((blk, 16), lambda i: (i, 0)),
            pl.BlockSpec((blk, 16), lambda i: (i, 0)),
        ],
        out_shape=[
            jax.ShapeDtypeStruct((N_NODES, D_OUT), jnp.float32),
            jax.ShapeDtypeStruct((N_NODES, 16), jnp.float32),
            jax.ShapeDtypeStruct((N_NODES, 16), jnp.float32),
        ],
    )(feat, wt, b1, b2)


def _rot8(v):
    perm = (lax.iota(jnp.int32, 16) + 8) % 16
    return lax.gather(
        v, perm[:, None],
        dimension_numbers=lax.GatherDimensionNumbers(
            offset_dims=(), collapsed_slice_dims=(0,), start_index_map=(0,)),
        slice_sizes=(1,), mode=lax.GatherScatterMode.PROMISE_IN_BOUNDS)


def _agg_body(fs_hbm, elr_hbm, rel_hbm, src_hbm, dst_hbm,
              es0_out, es1_out, rst0_out, rst1_out, ee_out, *scr):
    SRC = scr[0:3]
    DST = scr[3:6]
    RS = scr[6:8]
    RD = scr[8:10]
    EE = scr[10:12]
    EEP = scr[12:14]
    FSR = scr[14:17]
    esum_sh, rst_sh = scr[17], scr[18]
    gRS = scr[19:21]
    gRD = scr[21:23]
    gF = scr[23:26]
    sE = scr[26:28]
    sR = scr[28:31]
    sP = scr[31:33]

    cid = lax.axis_index("c")
    sid = lax.axis_index("s")
    wid = cid * NS + sid

    def _zero_fsr(i, _):
        e = i // N_HEADS
        h = i % N_HEADS
        FSR[0][e, pl.ds(h * D_HEAD, D_HEAD)] = jnp.zeros((16,), jnp.float32)
        return 0
    lax.fori_loop(0, CHUNK * N_HEADS, _zero_fsr, 0)

    def _zero_ee(i, _):
        EE[0][i] = jnp.zeros((16,), jnp.float32)
        return 0
    lax.fori_loop(0, CHUNK, _zero_ee, 0)

    r0 = sid * ROWS_PER_TILE
    for off in range(0, ROWS_PER_TILE, CHUNK):
        n = min(CHUNK, ROWS_PER_TILE - off)
        pltpu.sync_copy(FSR[0].at[pl.ds(0, n)], rst_sh.at[pl.ds(r0 + off, n)])
        pltpu.sync_copy(EE[0].at[pl.ds(0, n)], esum_sh.at[pl.ds(r0 + off, n)])

    @pl.when(sid == NS - 1)
    def _zero_tail():
        t0 = NS * ROWS_PER_TILE
        pltpu.sync_copy(FSR[0].at[pl.ds(0, ROWS_TAIL)],
                        rst_sh.at[pl.ds(t0, ROWS_TAIL)])
        pltpu.sync_copy(EE[0].at[pl.ds(0, ROWS_TAIL)],
                        esum_sh.at[pl.ds(t0, ROWS_TAIL)])

    plsc.subcore_barrier()

    mask = lax.iota(jnp.int32, 16) < N_HEADS

    def _issue(kk, p):
        si, s, f = p % 3, p % 2, p % 3
        base = wid * EPW + kk * CHUNK
        pltpu.sync_copy(src_hbm.at[pl.ds(base, CHUNK)], SRC[si])
        pltpu.sync_copy(dst_hbm.at[pl.ds(base, CHUNK)], DST[si])
        pltpu.async_copy(elr_hbm.at[SRC[si]], RS[s], gRS[s])
        pltpu.async_copy(rel_hbm.at[DST[si]], RD[s], gRD[s])
        pltpu.async_copy(fs_hbm.at[SRC[si]], FSR[f], gF[f])

    def _wait_se(p):
        si, s = p % 3, p % 2
        pltpu.make_async_copy(EE[s], esum_sh.at[DST[si]], sE[s]).wait()

    def _wait_sr(p):
        f = p % 3
        pltpu.make_async_copy(FSR[f], rst_sh.at[DST[f]], sR[f]).wait()

    def _wait_sp(kk, p):
        s = p % 2
        rbase = (wid * EPW + kk * CHUNK) // 2
        pltpu.make_async_copy(EEP[s], ee_out.at[pl.ds(rbase, CHUNK // 2)],
                              sP[s]).wait()

    def _process(kk, p):
        si, s, f = p % 3, p % 2, p % 3
        pltpu.make_async_copy(elr_hbm.at[SRC[si]], RS[s], gRS[s]).wait()
        pltpu.make_async_copy(rel_hbm.at[DST[si]], RD[s], gRD[s]).wait()
        pltpu.make_async_copy(fs_hbm.at[SRC[si]], FSR[f], gF[f]).wait()

        @plsc.parallel_loop(0, CHUNK, step=1, unroll=2)
        def _edge(e):
            v = RS[s][e] + RD[s][e]
            v = jnp.where(v >= 0, v, 0.2 * v)
            ee = jnp.where(mask, jnp.exp(v), 0.0)
            EE[s][e] = ee
            for h in range(N_HEADS):
                sc = ee[h]
                FSR[f][e, pl.ds(h * D_HEAD, D_HEAD)] = (
                    FSR[f][e, pl.ds(h * D_HEAD, D_HEAD)] * sc)

        pltpu.async_copy(EE[s], esum_sh.at[DST[si]], sE[s], add=True)

        @plsc.parallel_loop(0, CHUNK // 2, step=1, unroll=2)
        def _pair(i):
            EEP[s][i] = EE[s][2 * i] + _rot8(EE[s][2 * i + 1])

        rbase = (wid * EPW + kk * CHUNK) // 2
        pltpu.async_copy(EEP[s], ee_out.at[pl.ds(rbase, CHUNK // 2)], sP[s])
        pltpu.async_copy(FSR[f], rst_sh.at[DST[si]], sR[f], add=True)

    _issue(0, 0)

    def _six(k, _):
        for j in range(6):
            kk = 6 * k + j

            @pl.when(kk >= 2)
            def _w_rst():
                _wait_sr((j + 4) % 6)

            @pl.when(kk >= 1)
            def _w_es():
                _wait_se((j + 5) % 6)
                _wait_sp(kk - 1, (j + 5) % 6)

            _issue(kk + 1, (j + 1) % 6)
            _process(kk, j)
        return 0

    n_six = (NCHUNK - 5) // 6
    lax.fori_loop(0, n_six, _six, 0)

    for kk in range(6 * n_six, NCHUNK):
        _wait_sr((kk - 2) % 6)
        _wait_se((kk - 1) % 6)
        _wait_sp(kk - 1, (kk - 1) % 6)
        if kk < NCHUNK - 1:
            _issue(kk + 1, (kk + 1) % 6)
        _process(kk, kk % 6)

    _wait_se((NCHUNK - 1) % 6)
    _wait_sp(NCHUNK - 1, (NCHUNK - 1) % 6)
    _wait_sr((NCHUNK - 2) % 6)
    _wait_sr((NCHUNK - 1) % 6)

    plsc.subcore_barrier()

    def _dump(esum_out, rst_out):
        pltpu.sync_copy(esum_sh.at[pl.ds(r0, ROWS_PER_TILE)],
                        esum_out.at[pl.ds(r0, ROWS_PER_TILE)])
        pltpu.sync_copy(rst_sh.at[pl.ds(r0, ROWS_PER_TILE)],
                        rst_out.at[pl.ds(r0, ROWS_PER_TILE)])

        @pl.when(sid == NS - 1)
        def _dump_tail():
            t0 = NS * ROWS_PER_TILE
            pltpu.sync_copy(esum_sh.at[pl.ds(t0, ROWS_TAIL)],
                            esum_out.at[pl.ds(t0, ROWS_TAIL)])
            pltpu.sync_copy(rst_sh.at[pl.ds(t0, ROWS_TAIL)],
                            rst_out.at[pl.ds(t0, ROWS_TAIL)])

    @pl.when(cid == 0)
    def _d0():
        _dump(es0_out, rst0_out)

    @pl.when(cid == 1)
    def _d1():
        _dump(es1_out, rst1_out)


def _aggregate_stage(fs, elr, rel, src, dst):
    mesh = plsc.VectorSubcoreMesh(core_axis_name="c", subcore_axis_name="s",
                                  num_cores=NC, num_subcores=NS)
    f = pl.kernel(
        _agg_body,
        out_type=[
            jax.ShapeDtypeStruct((N_NODES, 16), jnp.float32),
            jax.ShapeDtypeStruct((N_NODES, 16), jnp.float32),
            jax.ShapeDtypeStruct((N_NODES, D_OUT), jnp.float32),
            jax.ShapeDtypeStruct((N_NODES, D_OUT), jnp.float32),
            jax.ShapeDtypeStruct((N_EDGES // 2, 16), jnp.float32),
        ],
        mesh=mesh,
        scratch_types=(
            [pltpu.VMEM((CHUNK,), jnp.int32)] * 6
            + [pltpu.VMEM((CHUNK, 16), jnp.float32)] * 6
            + [pltpu.VMEM((CHUNK // 2, 16), jnp.float32)] * 2
            + [pltpu.VMEM((CHUNK, D_OUT), jnp.float32)] * 3
            + [pltpu.VMEM_SHARED((N_NODES, 16), jnp.float32),
               pltpu.VMEM_SHARED((N_NODES, D_OUT), jnp.float32)]
            + [pltpu.SemaphoreType.DMA] * 14
        ),
        compiler_params=pltpu.CompilerParams(use_tc_tiling_on_sc=False),
    )
    return f(fs, elr, rel, src, dst)


def _attn_body(ee_hbm, es0_hbm, es1_hbm, dst_hbm, attn_out, *scr):
    DST = scr[0:2]
    ES0 = scr[2:4]
    ES1 = scr[4:6]
    EP = scr[6:8]
    AV = scr[8:10]
    gES0 = scr[10:12]
    gES1 = scr[12:14]
    gEP = scr[14:16]
    sA = scr[16:18]

    cid = lax.axis_index("c")
    sid = lax.axis_index("s")
    wid = cid * NS + sid
    HCH = CHUNK_B // 2

    def _issue(kk, b):
        base = wid * EPW + kk * CHUNK_B
        pltpu.sync_copy(dst_hbm.at[pl.ds(base, CHUNK_B)], DST[b])
        pltpu.async_copy(es0_hbm.at[DST[b]], ES0[b], gES0[b])
        pltpu.async_copy(es1_hbm.at[DST[b]], ES1[b], gES1[b])
        pltpu.async_copy(ee_hbm.at[pl.ds(base // 2, HCH)], EP[b], gEP[b])

    def _wait_sa(kk, b):
        base = wid * EPW + kk * CHUNK_B
        pltpu.make_async_copy(AV[b], attn_out.at[pl.ds(base // 2, HCH)],
                              sA[b]).wait()

    def _process(kk, b):
        pltpu.make_async_copy(es0_hbm.at[DST[b]], ES0[b], gES0[b]).wait()
        pltpu.make_async_copy(es1_hbm.at[DST[b]], ES1[b], gES1[b]).wait()
        base = wid * EPW + kk * CHUNK_B
        pltpu.make_async_copy(ee_hbm.at[pl.ds(base // 2, HCH)], EP[b],
                              gEP[b]).wait()

        @plsc.parallel_loop(0, HCH, step=1, unroll=2)
        def _pair(i):
            d0 = ES0[b][2 * i] + ES1[b][2 * i]
            d1 = ES0[b][2 * i + 1] + ES1[b][2 * i + 1]
            d = d0 + _rot8(d1) + 1e-9
            AV[b][i] = EP[b][i] / d

        pltpu.async_copy(AV[b], attn_out.at[pl.ds(base // 2, HCH)], sA[b])

    _issue(0, 0)

    def _pair(k, _):
        for j in range(2):
            kk = 2 * k + j
            b, b1 = j, 1 - j

            @pl.when(kk >= 1)
            def _w():
                _wait_sa(kk - 1, b1)

            _issue(kk + 1, b1)
            _process(kk, b)
        return 0

    n_pair = (NCHUNK_B - 1) // 2
    lax.fori_loop(0, n_pair, _pair, 0)

    for kk in range(2 * n_pair, NCHUNK_B):
        b = kk % 2
        if kk < NCHUNK_B - 1:
            _wait_sa(kk - 1, 1 - b)
            _issue(kk + 1, 1 - b)
        else:
            _wait_sa(kk - 1, 1 - b)
        _process(kk, b)
    _wait_sa(NCHUNK_B - 1, (NCHUNK_B - 1) % 2)


def _attn_stage(eep, es0, es1, dst):
    mesh = plsc.VectorSubcoreMesh(core_axis_name="c", subcore_axis_name="s",
                                  num_cores=NC, num_subcores=NS)
    f = pl.kernel(
        _attn_body,
        out_type=jax.ShapeDtypeStruct((N_EDGES // 2, 16), jnp.float32),
        mesh=mesh,
        scratch_types=(
            [pltpu.VMEM((CHUNK_B,), jnp.int32)] * 2
            + [pltpu.VMEM((CHUNK_B, 16), jnp.float32)] * 4
            + [pltpu.VMEM((CHUNK_B // 2, 16), jnp.float32)] * 4
            + [pltpu.SemaphoreType.DMA] * 8
        ),
        compiler_params=pltpu.CompilerParams(use_tc_tiling_on_sc=False),
    )
    return f(eep, es0, es1, dst)


def _norm_kernel(p0_ref, p1_ref, e0_ref, e1_ref, exp_ref, bias_ref,
                 out_ref, esc_ref):
    es = e0_ref[...] + e1_ref[...]
    esc_ref[...] = es
    denom = jnp.dot(es, exp_ref[...], preferred_element_type=jnp.float32,
                    precision=lax.Precision.HIGHEST) + 1e-9
    out_ref[...] = (p0_ref[...] + p1_ref[...]) / denom + bias_ref[...]


def _norm_stage(p0, p1, e0, e1, expand, bias_vec):
    blk = 1000
    grid = N_NODES // blk
    return pl.pallas_call(
        _norm_kernel,
        grid=(grid,),
        in_specs=[
            pl.BlockSpec((blk, D_OUT), lambda i: (i, 0)),
            pl.BlockSpec((blk, D_OUT), lambda i: (i, 0)),
            pl.BlockSpec((blk, 16), lambda i: (i, 0)),
            pl.BlockSpec((blk, 16), lambda i: (i, 0)),
            pl.BlockSpec((16, D_OUT), lambda i: (0, 0)),
            pl.BlockSpec((1, D_OUT), lambda i: (0, 0)),
        ],
        out_specs=[
            pl.BlockSpec((blk, D_OUT), lambda i: (i, 0)),
            pl.BlockSpec((blk, 16), lambda i: (i, 0)),
        ],
        out_shape=[
            jax.ShapeDtypeStruct((N_NODES, D_OUT), jnp.float32),
            jax.ShapeDtypeStruct((N_NODES, 16), jnp.float32),
        ],
    )(p0, p1, e0, e1, expand, bias_vec)


def _resh_kernel(in_ref, out_ref):
    out_ref[...] = in_ref[...].reshape(out_ref.shape)


def _reshape_stage(attn_pairs):
    blk = 8000
    grid = (N_EDGES // 2) // blk
    return pl.pallas_call(
        _resh_kernel,
        grid=(grid,),
        in_specs=[pl.BlockSpec((blk, 16), lambda i: (i, 0))],
        out_specs=pl.BlockSpec((2 * blk, 8), lambda i: (i, 0)),
        out_shape=jax.ShapeDtypeStruct((N_EDGES, 8), jnp.float32),
    )(attn_pairs)


@jax.jit
def _run(feat, edge_index, fc_weight, attn_l, attn_r, bias_param):
    src = edge_index[0].astype(jnp.int32)
    dst = edge_index[1].astype(jnp.int32)

    al = attn_l.reshape(N_HEADS, D_HEAD)
    ar = attn_r.reshape(N_HEADS, D_HEAD)
    hf = jnp.arange(D_OUT)
    head_of = hf // D_HEAD
    lane_of = hf % D_HEAD
    onehot = (head_of[:, None] == jnp.arange(N_HEADS)[None, :]).astype(jnp.float32)
    b_l = onehot * al[head_of, lane_of][:, None]
    b_r = onehot * ar[head_of, lane_of][:, None]
    b1 = jnp.concatenate([b_l, b_r], axis=1)
    b2 = jnp.concatenate([b_r, b_l], axis=1)

    fs, elr, rel = _linear_stage(feat, fc_weight.T, b1, b2)

    es0, es1, p0, p1, eep = _aggregate_stage(fs, elr, rel, src, dst)

    expand = (jnp.arange(16)[:, None] == head_of[None, :]).astype(jnp.float32)
    bias_vec = bias_param.reshape(1, D_OUT)
    rst, _ = _norm_stage(p0, p1, es0, es1, expand, bias_vec)

    attn_pairs = _attn_stage(eep, es0, es1, dst)

    return (rst.reshape(N_NODES, N_HEADS, D_HEAD),
            attn_pairs.reshape(N_EDGES, N_HEADS))


def kernel(feat, edge_index, e_feat, layer_idx, fc_weight, attn_l, attn_r, bias_param):
    del e_feat, layer_idx
    return _run(feat, edge_index, fc_weight, attn_l, attn_r, bias_param)


# speedup vs baseline: 1.0329x; 1.0003x over previous
import jax
import jax.numpy as jnp
from jax import lax
from jax.experimental import pallas as pl
from jax.experimental.pallas import tpu as pltpu
from jax.experimental.pallas import tpu_sc as plsc

N_NODES = 10000
N_EDGES = 320000
D_IN = 128
N_HEADS = 8
D_HEAD = 16
D_OUT = N_HEADS * D_HEAD

NC = 2
NS = 16
NW = NC * NS
EPW = N_EDGES // NW
CHUNK = 80
NCHUNK = EPW // CHUNK
CHUNK_B = 400
NCHUNK_B = EPW // CHUNK_B
ROWS_PER_TILE = 624
ROWS_TAIL = N_NODES - NS * ROWS_PER_TILE


def _lin_kernel(feat_ref, wt_ref, b1_ref, b2_ref, fs_ref, elr_ref, rel_ref):
    x = feat_ref[...]
    fs = jnp.dot(x, wt_ref[...], preferred_element_type=jnp.float32,
                 precision=lax.Precision.HIGHEST)
    fs_ref[...] = fs
    elr_ref[...] = jnp.dot(fs, b1_ref[...], preferred_element_type=jnp.float32,
                           precision=lax.Precision.HIGHEST)
    rel_ref[...] = jnp.dot(fs, b2_ref[...], preferred_element_type=jnp.float32,
                           precision=lax.Precision.HIGHEST)


def _linear_stage(feat, wt, b1, b2):
    blk = 1000
    grid = N_NODES // blk
    return pl.pallas_call(
        _lin_kernel,
        grid=(grid,),
        in_specs=[
            pl.BlockSpec((blk, D_IN), lambda i: (i, 0)),
            pl.BlockSpec((D_IN, D_OUT), lambda i: (0, 0)),
            pl.BlockSpec((D_OUT, 16), lambda i: (0, 0)),
            pl.BlockSpec((D_OUT, 16), lambda i: (0, 0)),
        ],
        out_specs=[
            pl.BlockSpec((blk, D_OUT), lambda i: (i, 0)),
            pl.BlockSpec((blk, 16), lambda i: (i, 0)),
            pl.BlockSpec((blk, 16), lambda i: (i, 0)),
        ],
        out_shape=[
            jax.ShapeDtypeStruct((N_NODES, D_OUT), jnp.float32),
            jax.ShapeDtypeStruct((N_NODES, 16), jnp.float32),
            jax.ShapeDtypeStruct((N_NODES, 16), jnp.float32),
        ],
    )(feat, wt, b1, b2)


def _rot8(v):
    perm = (lax.iota(jnp.int32, 16) + 8) % 16
    return lax.gather(
        v, perm[:, None],
        dimension_numbers=lax.GatherDimensionNumbers(
            offset_dims=(), collapsed_slice_dims=(0,), start_index_map=(0,)),
        slice_sizes=(1,), mode=lax.GatherScatterMode.PROMISE_IN_BOUNDS)


def _agg_body(fs_hbm, elr_hbm, rel_hbm, src_hbm, dst_hbm,
              es0_out, es1_out, rst0_out, rst1_out, ee_out, *scr):
    SRC = scr[0:3]
    DST = scr[3:6]
    RS = scr[6:8]
    RD = scr[8:10]
    EE = scr[10:12]
    EEP = scr[12:14]
    FSR = scr[14:17]
    esum_sh, rst_sh = scr[17], scr[18]
    gRS = scr[19:21]
    gRD = scr[21:23]
    gF = scr[23:26]
    sE = scr[26:28]
    sR = scr[28:31]
    sP = scr[31:33]

    cid = lax.axis_index("c")
    sid = lax.axis_index("s")
    wid = cid * NS + sid

    def _zero_fsr(i, _):
        e = i // N_HEADS
        h = i % N_HEADS
        FSR[0][e, pl.ds(h * D_HEAD, D_HEAD)] = jnp.zeros((16,), jnp.float32)
        return 0
    lax.fori_loop(0, CHUNK * N_HEADS, _zero_fsr, 0)

    def _zero_ee(i, _):
        EE[0][i] = jnp.zeros((16,), jnp.float32)
        return 0
    lax.fori_loop(0, CHUNK, _zero_ee, 0)

    r0 = sid * ROWS_PER_TILE
    for off in range(0, ROWS_PER_TILE, CHUNK):
        n = min(CHUNK, ROWS_PER_TILE - off)
        pltpu.sync_copy(FSR[0].at[pl.ds(0, n)], rst_sh.at[pl.ds(r0 + off, n)])
        pltpu.sync_copy(EE[0].at[pl.ds(0, n)], esum_sh.at[pl.ds(r0 + off, n)])

    @pl.when(sid == NS - 1)
    def _zero_tail():
        t0 = NS * ROWS_PER_TILE
        pltpu.sync_copy(FSR[0].at[pl.ds(0, ROWS_TAIL)],
                        rst_sh.at[pl.ds(t0, ROWS_TAIL)])
        pltpu.sync_copy(EE[0].at[pl.ds(0, ROWS_TAIL)],
                        esum_sh.at[pl.ds(t0, ROWS_TAIL)])

    plsc.subcore_barrier()

    mask = lax.iota(jnp.int32, 16) < N_HEADS

    def _issue(kk, p):
        si, s, f = p % 3, p % 2, p % 3
        base = wid * EPW + kk * CHUNK
        pltpu.sync_copy(src_hbm.at[pl.ds(base, CHUNK)], SRC[si])
        pltpu.sync_copy(dst_hbm.at[pl.ds(base, CHUNK)], DST[si])
        pltpu.async_copy(elr_hbm.at[SRC[si]], RS[s], gRS[s])
        pltpu.async_copy(rel_hbm.at[DST[si]], RD[s], gRD[s])
        pltpu.async_copy(fs_hbm.at[SRC[si]], FSR[f], gF[f])

    def _wait_se(p):
        si, s = p % 3, p % 2
        pltpu.make_async_copy(EE[s], esum_sh.at[DST[si]], sE[s]).wait()

    def _wait_sr(p):
        f = p % 3
        pltpu.make_async_copy(FSR[f], rst_sh.at[DST[f]], sR[f]).wait()

    def _wait_sp(kk, p):
        s = p % 2
        rbase = (wid * EPW + kk * CHUNK) // 2
        pltpu.make_async_copy(EEP[s], ee_out.at[pl.ds(rbase, CHUNK // 2)],
                              sP[s]).wait()

    def _process(kk, p):
        si, s, f = p % 3, p % 2, p % 3
        pltpu.make_async_copy(elr_hbm.at[SRC[si]], RS[s], gRS[s]).wait()
        pltpu.make_async_copy(rel_hbm.at[DST[si]], RD[s], gRD[s]).wait()
        pltpu.make_async_copy(fs_hbm.at[SRC[si]], FSR[f], gF[f]).wait()

        @plsc.parallel_loop(0, CHUNK, step=1, unroll=2)
        def _edge(e):
            v = RS[s][e] + RD[s][e]
            v = jnp.where(v >= 0, v, 0.2 * v)
            ee = jnp.where(mask, jnp.exp(v), 0.0)
            EE[s][e] = ee
            for h in range(N_HEADS):
                sc = ee[h]
                FSR[f][e, pl.ds(h * D_HEAD, D_HEAD)] = (
                    FSR[f][e, pl.ds(h * D_HEAD, D_HEAD)] * sc)

        pltpu.async_copy(EE[s], esum_sh.at[DST[si]], sE[s], add=True)

        @plsc.parallel_loop(0, CHUNK // 2, step=1, unroll=2)
        def _pair(i):
            EEP[s][i] = EE[s][2 * i] + _rot8(EE[s][2 * i + 1])

        rbase = (wid * EPW + kk * CHUNK) // 2
        pltpu.async_copy(EEP[s], ee_out.at[pl.ds(rbase, CHUNK // 2)], sP[s])
        pltpu.async_copy(FSR[f], rst_sh.at[DST[si]], sR[f], add=True)

    _issue(0, 0)

    def _six(k, _):
        for j in range(6):
            kk = 6 * k + j

            @pl.when(kk >= 2)
            def _w_rst():
                _wait_sr((j + 4) % 6)

            @pl.when(kk >= 1)
            def _w_es():
                _wait_se((j + 5) % 6)
                _wait_sp(kk - 1, (j + 5) % 6)

            _issue(kk + 1, (j + 1) % 6)
            _process(kk, j)
        return 0

    n_six = (NCHUNK - 5) // 6
    lax.fori_loop(0, n_six, _six, 0)

    for kk in range(6 * n_six, NCHUNK):
        _wait_sr((kk - 2) % 6)
        _wait_se((kk - 1) % 6)
        _wait_sp(kk - 1, (kk - 1) % 6)
        if kk < NCHUNK - 1:
            _issue(kk + 1, (kk + 1) % 6)
        _process(kk, kk % 6)

    _wait_se((NCHUNK - 1) % 6)
    _wait_sp(NCHUNK - 1, (NCHUNK - 1) % 6)
    _wait_sr((NCHUNK - 2) % 6)
    _wait_sr((NCHUNK - 1) % 6)

    plsc.subcore_barrier()

    def _dump(esum_out, rst_out):
        pltpu.sync_copy(esum_sh.at[pl.ds(r0, ROWS_PER_TILE)],
                        esum_out.at[pl.ds(r0, ROWS_PER_TILE)])
        pltpu.sync_copy(rst_sh.at[pl.ds(r0, ROWS_PER_TILE)],
                        rst_out.at[pl.ds(r0, ROWS_PER_TILE)])

        @pl.when(sid == NS - 1)
        def _dump_tail():
            t0 = NS * ROWS_PER_TILE
            pltpu.sync_copy(esum_sh.at[pl.ds(t0, ROWS_TAIL)],
                            esum_out.at[pl.ds(t0, ROWS_TAIL)])
            pltpu.sync_copy(rst_sh.at[pl.ds(t0, ROWS_TAIL)],
                            rst_out.at[pl.ds(t0, ROWS_TAIL)])

    @pl.when(cid == 0)
    def _d0():
        _dump(es0_out, rst0_out)

    @pl.when(cid == 1)
    def _d1():
        _dump(es1_out, rst1_out)


def _aggregate_stage(fs, elr, rel, src, dst):
    mesh = plsc.VectorSubcoreMesh(core_axis_name="c", subcore_axis_name="s",
                                  num_cores=NC, num_subcores=NS)
    f = pl.kernel(
        _agg_body,
        out_type=[
            jax.ShapeDtypeStruct((N_NODES, 16), jnp.float32),
            jax.ShapeDtypeStruct((N_NODES, 16), jnp.float32),
            jax.ShapeDtypeStruct((N_NODES, D_OUT), jnp.float32),
            jax.ShapeDtypeStruct((N_NODES, D_OUT), jnp.float32),
            jax.ShapeDtypeStruct((N_EDGES // 2, 16), jnp.float32),
        ],
        mesh=mesh,
        scratch_types=(
            [pltpu.VMEM((CHUNK,), jnp.int32)] * 6
            + [pltpu.VMEM((CHUNK, 16), jnp.float32)] * 6
            + [pltpu.VMEM((CHUNK // 2, 16), jnp.float32)] * 2
            + [pltpu.VMEM((CHUNK, D_OUT), jnp.float32)] * 3
            + [pltpu.VMEM_SHARED((N_NODES, 16), jnp.float32),
               pltpu.VMEM_SHARED((N_NODES, D_OUT), jnp.float32)]
            + [pltpu.SemaphoreType.DMA] * 14
        ),
        compiler_params=pltpu.CompilerParams(use_tc_tiling_on_sc=False),
    )
    return f(fs, elr, rel, src, dst)


def _attn_body(ee_hbm, es0_hbm, es1_hbm, dst_hbm, attn_out, *scr):
    DST = scr[0:2]
    ES0 = scr[2:4]
    ES1 = scr[4:6]
    EP = scr[6:8]
    AV = scr[8:10]
    gES0 = scr[10:12]
    gES1 = scr[12:14]
    gEP = scr[14:16]
    sA = scr[16:18]

    cid = lax.axis_index("c")
    sid = lax.axis_index("s")
    wid = cid * NS + sid
    HCH = CHUNK_B // 2

    def _issue(kk, b):
        base = wid * EPW + kk * CHUNK_B
        pltpu.sync_copy(dst_hbm.at[pl.ds(base, CHUNK_B)], DST[b])
        pltpu.async_copy(es0_hbm.at[DST[b]], ES0[b], gES0[b])
        pltpu.async_copy(es1_hbm.at[DST[b]], ES1[b], gES1[b])
        pltpu.async_copy(ee_hbm.at[pl.ds(base // 2, HCH)], EP[b], gEP[b])

    def _wait_sa(kk, b):
        base = wid * EPW + kk * CHUNK_B
        pltpu.make_async_copy(AV[b], attn_out.at[pl.ds(base // 2, HCH)],
                              sA[b]).wait()

    def _process(kk, b):
        pltpu.make_async_copy(es0_hbm.at[DST[b]], ES0[b], gES0[b]).wait()
        pltpu.make_async_copy(es1_hbm.at[DST[b]], ES1[b], gES1[b]).wait()
        base = wid * EPW + kk * CHUNK_B
        pltpu.make_async_copy(ee_hbm.at[pl.ds(base // 2, HCH)], EP[b],
                              gEP[b]).wait()

        @plsc.parallel_loop(0, HCH, step=1, unroll=2)
        def _pair(i):
            d0 = ES0[b][2 * i] + ES1[b][2 * i]
            d1 = ES0[b][2 * i + 1] + ES1[b][2 * i + 1]
            d = d0 + _rot8(d1) + 1e-9
            AV[b][i] = EP[b][i] / d

        pltpu.async_copy(AV[b], attn_out.at[pl.ds(base // 2, HCH)], sA[b])

    _issue(0, 0)

    def _pair(k, _):
        for j in range(2):
            kk = 2 * k + j
            b, b1 = j, 1 - j

            @pl.when(kk >= 1)
            def _w():
                _wait_sa(kk - 1, b1)

            _issue(kk + 1, b1)
            _process(kk, b)
        return 0

    n_pair = (NCHUNK_B - 1) // 2
    lax.fori_loop(0, n_pair, _pair, 0)

    for kk in range(2 * n_pair, NCHUNK_B):
        b = kk % 2
        if kk < NCHUNK_B - 1:
            _wait_sa(kk - 1, 1 - b)
            _issue(kk + 1, 1 - b)
        else:
            _wait_sa(kk - 1, 1 - b)
        _process(kk, b)
    _wait_sa(NCHUNK_B - 1, (NCHUNK_B - 1) % 2)


def _attn_stage(eep, es0, es1, dst):
    mesh = plsc.VectorSubcoreMesh(core_axis_name="c", subcore_axis_name="s",
                                  num_cores=NC, num_subcores=NS)
    f = pl.kernel(
        _attn_body,
        out_type=jax.ShapeDtypeStruct((N_EDGES // 2, 16), jnp.float32),
        mesh=mesh,
        scratch_types=(
            [pltpu.VMEM((CHUNK_B,), jnp.int32)] * 2
            + [pltpu.VMEM((CHUNK_B, 16), jnp.float32)] * 4
            + [pltpu.VMEM((CHUNK_B // 2, 16), jnp.float32)] * 4
            + [pltpu.SemaphoreType.DMA] * 8
        ),
        compiler_params=pltpu.CompilerParams(use_tc_tiling_on_sc=False),
    )
    return f(eep, es0, es1, dst)


def _norm_kernel(p0_ref, p1_ref, e0_ref, e1_ref, exp_ref, bias_ref,
                 out_ref, esc_ref):
    es = e0_ref[...] + e1_ref[...]
    esc_ref[...] = es
    denom = jnp.dot(es, exp_ref[...], preferred_element_type=jnp.float32,
                    precision=lax.Precision.HIGHEST) + 1e-9
    out_ref[...] = (p0_ref[...] + p1_ref[...]) / denom + bias_ref[...]


def _norm_stage(p0, p1, e0, e1, expand, bias_vec):
    blk = 1000
    grid = N_NODES // blk
    return pl.pallas_call(
        _norm_kernel,
        grid=(grid,),
        in_specs=[
            pl.BlockSpec((blk, D_OUT), lambda i: (i, 0)),
            pl.BlockSpec((blk, D_OUT), lambda i: (i, 0)),
            pl.BlockSpec((blk, 16), lambda i: (i, 0)),
            pl.BlockSpec((blk, 16), lambda i: (i, 0)),
            pl.BlockSpec((16, D_OUT), lambda i: (0, 0)),
            pl.BlockSpec((1, D_OUT), lambda i: (0, 0)),
        ],
        out_specs=[
            pl.BlockSpec((blk, D_OUT), lambda i: (i, 0)),
            pl.BlockSpec((blk, 16), lambda i: (i, 0)),
        ],
        out_shape=[
            jax.ShapeDtypeStruct((N_NODES, D_OUT), jnp.float32),
            jax.ShapeDtypeStruct((N_NODES, 16), jnp.float32),
        ],
    )(p0, p1, e0, e1, expand, bias_vec)


@jax.jit
def _run(feat, edge_index, fc_weight, attn_l, attn_r, bias_param):
    src = edge_index[0].astype(jnp.int32)
    dst = edge_index[1].astype(jnp.int32)

    al = attn_l.reshape(N_HEADS, D_HEAD)
    ar = attn_r.reshape(N_HEADS, D_HEAD)
    hf = jnp.arange(D_OUT)
    head_of = hf // D_HEAD
    lane_of = hf % D_HEAD
    onehot = (head_of[:, None] == jnp.arange(N_HEADS)[None, :]).astype(jnp.float32)
    b_l = onehot * al[head_of, lane_of][:, None]
    b_r = onehot * ar[head_of, lane_of][:, None]
    b1 = jnp.concatenate([b_l, b_r], axis=1)
    b2 = jnp.concatenate([b_r, b_l], axis=1)

    fs, elr, rel = _linear_stage(feat, fc_weight.T, b1, b2)

    es0, es1, p0, p1, eep = _aggregate_stage(fs, elr, rel, src, dst)

    expand = (jnp.arange(16)[:, None] == head_of[None, :]).astype(jnp.float32)
    bias_vec = bias_param.reshape(1, D_OUT)
    rst, _ = _norm_stage(p0, p1, es0, es1, expand, bias_vec)

    attn_pairs = _attn_stage(eep, es0, es1, dst)

    return (rst.reshape(N_NODES, N_HEADS, D_HEAD),
            attn_pairs.reshape(N_EDGES, N_HEADS))


def kernel(feat, edge_index, e_feat, layer_idx, fc_weight, attn_l, attn_r, bias_param):
    del e_feat, layer_idx
    return _run(feat, edge_index, fc_weight, attn_l, attn_r, bias_param)
